# baseline (speedup 1.0000x reference)
import functools

import jax
import jax.numpy as jnp
from jax import lax
from jax.experimental import pallas as pl
from jax.experimental.pallas import tpu as pltpu
from jax.experimental.pallas import tpu_sc as plsc

N = 10000
E = 320000
R = 8
D = 128
B = 1024

NC = 2
NS = 16
NW = NC * NS
EPT = E // NS
SB = 2000
L = 16

CG = 128
HB = B // 2
NSEG = HB * R
DUMP = NSEG
NSEG_PAD = NSEG + 16
CPAD = 4224
CBUF = SB + 3 * CG
SROWS = B // NW


def _sc_body(inv_hbm, src_hbm, dstt_hbm, nodes_hbm, feat_hbm,
             ones_hbm, zrow_hbm, zcnt_hbm,
             acc_out, cnt_out, self_out,
             inv_v, src_v, dstt_v, csrc_v, cseg_v,
             src_row, seg_row, rows_v, onesb, zcnt_v, nidx_v,
             acc_sh, cnt_sh, sem):
    c = lax.axis_index("c")
    s = lax.axis_index("s")
    wid = c * NS + s

    pltpu.sync_copy(inv_hbm, inv_v)
    pltpu.sync_copy(ones_hbm, onesb)
    pltpu.sync_copy(zrow_hbm, rows_v)
    pltpu.sync_copy(zcnt_hbm, zcnt_v)

    for q in range(2):
        pltpu.sync_copy(rows_v, acc_sh.at[pl.ds(s * 257 + q * CG, CG)])
    pltpu.sync_copy(rows_v.at[pl.ds(0, 1)], acc_sh.at[pl.ds(s * 257 + 256, 1)])
    pltpu.sync_copy(zcnt_v, cnt_sh.at[pl.ds(s * 264, 264)])
    plsc.subcore_barrier()

    def p1(i, cnt):
        base = i * L
        dstt16 = dstt_v[pl.ds(base, L)]
        src16 = src_v[pl.ds(base, L)]
        dst16 = lax.shift_right_logical(dstt16, 3)
        typ16 = dstt16 & 7
        w16 = plsc.load_gather(inv_v, [lax.shift_right_logical(dst16, 1)])
        sv16 = lax.shift_right_logical(w16, (dst16 & 1) * 16) & 0xFFFF
        slot16 = sv16 - 1
        m = (sv16 > 0) & ((slot16 & 1) == c)
        seg16 = jnp.where(m, lax.shift_right_logical(slot16, 1) * R + typ16,
                          DUMP)
        plsc.store_compressed(csrc_v.at[pl.ds(cnt, L)], src16, mask=m)
        plsc.store_compressed(cseg_v.at[pl.ds(cnt, L)], seg16, mask=m)
        return cnt + jnp.sum(m.astype(jnp.int32))

    def p2(k, _):
        for j in range(CG // L):
            src_row[pl.ds(j * L, L)] = csrc_v[pl.ds(k * CG + j * L, L)]
            seg_row[pl.ds(j * L, L)] = cseg_v[pl.ds(k * CG + j * L, L)]
        pltpu.async_copy(feat_hbm.at[src_row], rows_v, sem).wait()
        pltpu.sync_copy(rows_v, acc_sh.at[seg_row], add=True)
        pltpu.sync_copy(onesb, cnt_sh.at[seg_row], add=True)
        return 0

    def sub(t, cnt):
        off = s * EPT + t * SB
        d0 = pltpu.async_copy(src_hbm.at[pl.ds(off, SB)], src_v, sem)
        d1 = pltpu.async_copy(dstt_hbm.at[pl.ds(off, SB)], dstt_v, sem)
        d0.wait(); d1.wait()
        cnt = lax.fori_loop(0, SB // L, p1, cnt)
        nfull = cnt // CG
        lax.fori_loop(0, nfull, p2, 0)
        for j in range(CG // L):
            sv = csrc_v[pl.ds(nfull * CG + j * L, L)]
            gv = cseg_v[pl.ds(nfull * CG + j * L, L)]
            csrc_v[pl.ds(j * L, L)] = sv
            cseg_v[pl.ds(j * L, L)] = gv
        return cnt - nfull * CG

    cnt = lax.fori_loop(0, EPT // SB, sub, jnp.int32(0))

    full = jnp.ones((L,), jnp.bool_)
    for k in range(CG // L):
        plsc.store_compressed(
            cseg_v.at[pl.ds(cnt + k * L, L)],
            jnp.full((L,), DUMP, jnp.int32), mask=full)
        plsc.store_compressed(
            csrc_v.at[pl.ds(cnt + k * L, L)],
            jnp.zeros((L,), jnp.int32), mask=full)
    lax.fori_loop(0, 1, p2, 0)
    plsc.subcore_barrier()

    pltpu.sync_copy(acc_sh.at[pl.ds(s * 256, 256)],
                    acc_out.at[c, pl.ds(s * 256, 256)])
    pltpu.sync_copy(cnt_sh.at[pl.ds(s * 256, 256)],
                    cnt_out.at[c, pl.ds(s * 256, 256)])

    pltpu.sync_copy(nodes_hbm.at[pl.ds(wid * SROWS, SROWS)], nidx_v)
    pltpu.async_copy(feat_hbm.at[nidx_v], rows_v.at[pl.ds(0, SROWS)],
                     sem).wait()
    pltpu.sync_copy(rows_v.at[pl.ds(0, SROWS)],
                    self_out.at[pl.ds(wid * SROWS, SROWS)])


_sc_agg = functools.partial(
    pl.kernel,
    out_type=[
        jax.ShapeDtypeStruct((NC, NSEG, D), jnp.float32),
        jax.ShapeDtypeStruct((NC, NSEG), jnp.float32),
        jax.ShapeDtypeStruct((B, D), jnp.float32),
    ],
    mesh=plsc.VectorSubcoreMesh(
        core_axis_name="c", subcore_axis_name="s",
        num_cores=NC, num_subcores=NS),
    scratch_types=[
        pltpu.VMEM((N // 2,), jnp.int32),
        pltpu.VMEM((SB,), jnp.int32),
        pltpu.VMEM((SB,), jnp.int32),
        pltpu.VMEM((CBUF,), jnp.int32),
        pltpu.VMEM((CBUF,), jnp.int32),
        pltpu.VMEM((CG,), jnp.int32),
        pltpu.VMEM((CG,), jnp.int32),
        pltpu.VMEM((CG, D), jnp.float32),
        pltpu.VMEM((CG,), jnp.float32),
        pltpu.VMEM((264,), jnp.float32),
        pltpu.VMEM((SROWS,), jnp.int32),
        pltpu.VMEM_SHARED((NSEG_PAD, D), jnp.float32),
        pltpu.VMEM_SHARED((CPAD,), jnp.float32),
        pltpu.SemaphoreType.DMA,
    ],
    compiler_params=pltpu.CompilerParams(needs_layout_passes=False),
)(_sc_body)


BT = 256


def _tc_body(acc_ref, cnt_ref, self_ref, w_ref, rw_ref, out_ref):
    acc = acc_ref[...]
    cnt = cnt_ref[...]
    rel = jnp.zeros((BT, D), jnp.float32)
    for r in range(R):
        mean_r = acc[:, r, :] / (cnt[:, r:r + 1] + 1e-10)
        rel = rel + lax.dot_general(
            mean_r, rw_ref[r],
            (((1,), (1,)), ((), ())), preferred_element_type=jnp.float32)
    self_o = lax.dot_general(
        self_ref[...], w_ref[...],
        (((1,), (1,)), ((), ())), preferred_element_type=jnp.float32)
    out_ref[...] = jnp.maximum(self_o + rel, 0.0)


def _tc_combine(acc3, cnt2, self_rows, weight, relation_weights):
    return pl.pallas_call(
        _tc_body,
        grid=(B // BT,),
        in_specs=[
            pl.BlockSpec((BT, R, D), lambda i: (i, 0, 0)),
            pl.BlockSpec((BT, R), lambda i: (i, 0)),
            pl.BlockSpec((BT, D), lambda i: (i, 0)),
            pl.BlockSpec((D, D), lambda i: (0, 0)),
            pl.BlockSpec((R, D, D), lambda i: (0, 0, 0)),
        ],
        out_specs=pl.BlockSpec((BT, D), lambda i: (i, 0)),
        out_shape=jax.ShapeDtypeStruct((B, D), jnp.float32),
    )(acc3, cnt2, self_rows, weight, relation_weights)


def kernel(nodes, features, edge_index, edge_type, weight, relation_weights):
    nodes = nodes.astype(jnp.int32)
    src = edge_index[0].astype(jnp.int32)
    dstt = edge_index[1].astype(jnp.int32) * 8 + edge_type.astype(jnp.int32)

    inv = jnp.full((N,), -1, jnp.int32).at[nodes].set(
        jnp.arange(B, dtype=jnp.int32))
    svp = inv + 1
    inv_packed = svp[0::2] | (svp[1::2] << 16)

    nodes_perm = jnp.transpose(nodes.reshape(HB, 2)).reshape(B)

    ones_in = jnp.ones((CG,), jnp.float32)
    zrow = jnp.zeros((CG, D), jnp.float32)
    zcnt = jnp.zeros((264,), jnp.float32)

    acc, cnt, self_rows = _sc_agg(
        inv_packed, src, dstt, nodes_perm, features, ones_in, zrow, zcnt)

    out_perm = _tc_combine(
        acc.reshape(B, R, D), cnt.reshape(B, R),
        self_rows, weight, relation_weights)

    canon = inv[nodes]
    return jnp.take(out_perm, (canon % 2) * HB + canon // 2, axis=0)


# speedup vs baseline: 1.1956x; 1.1956x over previous
import functools

import jax
import jax.numpy as jnp
from jax import lax
from jax.experimental import pallas as pl
from jax.experimental.pallas import tpu as pltpu
from jax.experimental.pallas import tpu_sc as plsc

N = 10000
E = 320000
R = 8
D = 128
B = 1024

NC = 2
NS = 16
NW = NC * NS
EPT = E // NS
SB = 2000
L = 16

CG = 64
HB = B // 2
NSEG = HB * R
DUMP = NSEG
NSEG_PAD = NSEG + 16
CPAD = 4224
CBUF = SB + 3 * CG
SROWS = B // NW


def _sc_body(inv_hbm, src_hbm, dstt_hbm, nodes_hbm, feat_hbm,
             ones_hbm, zrow_hbm, zcnt_hbm,
             acc_out, cnt_out, self_out,
             inv_v, src_v, dstt_v, csrc_v, cseg_v,
             src_row, seg_row, rows_v, onesb, zcnt_v, nidx_v,
             acc_sh, cnt_sh, sem):
    c = lax.axis_index("c")
    s = lax.axis_index("s")
    wid = c * NS + s

    pltpu.sync_copy(inv_hbm, inv_v)
    pltpu.sync_copy(ones_hbm, onesb)
    pltpu.sync_copy(zrow_hbm, rows_v)
    pltpu.sync_copy(zcnt_hbm, zcnt_v)

    for q in range(4):
        pltpu.sync_copy(rows_v, acc_sh.at[pl.ds(s * 257 + q * CG, CG)])
    pltpu.sync_copy(rows_v.at[pl.ds(0, 1)], acc_sh.at[pl.ds(s * 257 + 256, 1)])
    pltpu.sync_copy(zcnt_v, cnt_sh.at[pl.ds(s * 264, 264)])
    plsc.subcore_barrier()

    def p1(i, cnt):
        base = i * L
        dstt16 = dstt_v[pl.ds(base, L)]
        src16 = src_v[pl.ds(base, L)]
        dst16 = lax.shift_right_logical(dstt16, 3)
        typ16 = dstt16 & 7
        w16 = plsc.load_gather(inv_v, [lax.shift_right_logical(dst16, 1)])
        sv16 = lax.shift_right_logical(w16, (dst16 & 1) * 16) & 0xFFFF
        slot16 = sv16 - 1
        m = (sv16 > 0) & ((slot16 & 1) == c)
        seg16 = jnp.where(m, lax.shift_right_logical(slot16, 1) * R + typ16,
                          DUMP)
        plsc.store_compressed(csrc_v.at[pl.ds(cnt, L)], src16, mask=m)
        plsc.store_compressed(cseg_v.at[pl.ds(cnt, L)], seg16, mask=m)
        return cnt + jnp.sum(m.astype(jnp.int32))

    def p2(k, _):
        for j in range(CG // L):
            src_row[pl.ds(j * L, L)] = csrc_v[pl.ds(k * CG + j * L, L)]
            seg_row[pl.ds(j * L, L)] = cseg_v[pl.ds(k * CG + j * L, L)]
        pltpu.async_copy(feat_hbm.at[src_row], rows_v, sem).wait()
        pltpu.sync_copy(rows_v, acc_sh.at[seg_row], add=True)
        pltpu.sync_copy(onesb, cnt_sh.at[seg_row], add=True)
        return 0

    def sub(t, cnt):
        off = s * EPT + t * SB
        d0 = pltpu.async_copy(src_hbm.at[pl.ds(off, SB)], src_v, sem)
        d1 = pltpu.async_copy(dstt_hbm.at[pl.ds(off, SB)], dstt_v, sem)
        d0.wait(); d1.wait()
        cnt = lax.fori_loop(0, SB // L, p1, cnt)
        nfull = cnt // CG
        lax.fori_loop(0, nfull, p2, 0)
        for j in range(CG // L):
            sv = csrc_v[pl.ds(nfull * CG + j * L, L)]
            gv = cseg_v[pl.ds(nfull * CG + j * L, L)]
            csrc_v[pl.ds(j * L, L)] = sv
            cseg_v[pl.ds(j * L, L)] = gv
        return cnt - nfull * CG

    cnt = lax.fori_loop(0, EPT // SB, sub, jnp.int32(0))

    full = jnp.ones((L,), jnp.bool_)
    for k in range(CG // L):
        plsc.store_compressed(
            cseg_v.at[pl.ds(cnt + k * L, L)],
            jnp.full((L,), DUMP, jnp.int32), mask=full)
        plsc.store_compressed(
            csrc_v.at[pl.ds(cnt + k * L, L)],
            jnp.zeros((L,), jnp.int32), mask=full)
    lax.fori_loop(0, 1, p2, 0)
    plsc.subcore_barrier()

    pltpu.sync_copy(acc_sh.at[pl.ds(s * 256, 256)],
                    acc_out.at[c, pl.ds(s * 256, 256)])
    pltpu.sync_copy(cnt_sh.at[pl.ds(s * 256, 256)],
                    cnt_out.at[c, pl.ds(s * 256, 256)])

    pltpu.sync_copy(nodes_hbm.at[pl.ds(wid * SROWS, SROWS)], nidx_v)
    pltpu.async_copy(feat_hbm.at[nidx_v], rows_v.at[pl.ds(0, SROWS)],
                     sem).wait()
    pltpu.sync_copy(rows_v.at[pl.ds(0, SROWS)],
                    self_out.at[pl.ds(wid * SROWS, SROWS)])


_sc_agg = functools.partial(
    pl.kernel,
    out_type=[
        jax.ShapeDtypeStruct((NC, NSEG, D), jnp.float32),
        jax.ShapeDtypeStruct((NC, NSEG), jnp.float32),
        jax.ShapeDtypeStruct((B, D), jnp.float32),
    ],
    mesh=plsc.VectorSubcoreMesh(
        core_axis_name="c", subcore_axis_name="s",
        num_cores=NC, num_subcores=NS),
    scratch_types=[
        pltpu.VMEM((N // 2,), jnp.int32),
        pltpu.VMEM((SB,), jnp.int32),
        pltpu.VMEM((SB,), jnp.int32),
        pltpu.VMEM((CBUF,), jnp.int32),
        pltpu.VMEM((CBUF,), jnp.int32),
        pltpu.VMEM((CG,), jnp.int32),
        pltpu.VMEM((CG,), jnp.int32),
        pltpu.VMEM((CG, D), jnp.float32),
        pltpu.VMEM((CG,), jnp.float32),
        pltpu.VMEM((264,), jnp.float32),
        pltpu.VMEM((SROWS,), jnp.int32),
        pltpu.VMEM_SHARED((NSEG_PAD, D), jnp.float32),
        pltpu.VMEM_SHARED((CPAD,), jnp.float32),
        pltpu.SemaphoreType.DMA,
    ],
    compiler_params=pltpu.CompilerParams(needs_layout_passes=False),
)(_sc_body)


BT = 256


def _tc_body(acc_ref, cnt_ref, self_ref, w_ref, rw_ref, out_ref):
    acc = acc_ref[...]
    cnt = cnt_ref[...]
    rel = jnp.zeros((BT, D), jnp.float32)
    for r in range(R):
        mean_r = acc[:, r, :] / (cnt[:, r:r + 1] + 1e-10)
        rel = rel + lax.dot_general(
            mean_r, rw_ref[r],
            (((1,), (1,)), ((), ())), preferred_element_type=jnp.float32)
    self_o = lax.dot_general(
        self_ref[...], w_ref[...],
        (((1,), (1,)), ((), ())), preferred_element_type=jnp.float32)
    out_ref[...] = jnp.maximum(self_o + rel, 0.0)


def _tc_combine(acc3, cnt2, self_rows, weight, relation_weights):
    return pl.pallas_call(
        _tc_body,
        grid=(B // BT,),
        in_specs=[
            pl.BlockSpec((BT, R, D), lambda i: (i, 0, 0)),
            pl.BlockSpec((BT, R), lambda i: (i, 0)),
            pl.BlockSpec((BT, D), lambda i: (i, 0)),
            pl.BlockSpec((D, D), lambda i: (0, 0)),
            pl.BlockSpec((R, D, D), lambda i: (0, 0, 0)),
        ],
        out_specs=pl.BlockSpec((BT, D), lambda i: (i, 0)),
        out_shape=jax.ShapeDtypeStruct((B, D), jnp.float32),
    )(acc3, cnt2, self_rows, weight, relation_weights)


def kernel(nodes, features, edge_index, edge_type, weight, relation_weights):
    nodes = nodes.astype(jnp.int32)
    src = edge_index[0].astype(jnp.int32)
    dstt = edge_index[1].astype(jnp.int32) * 8 + edge_type.astype(jnp.int32)

    inv = jnp.full((N,), -1, jnp.int32).at[nodes].set(
        jnp.arange(B, dtype=jnp.int32))
    svp = inv + 1
    inv_packed = svp[0::2] | (svp[1::2] << 16)

    nodes_perm = jnp.transpose(nodes.reshape(HB, 2)).reshape(B)

    ones_in = jnp.ones((CG,), jnp.float32)
    zrow = jnp.zeros((CG, D), jnp.float32)
    zcnt = jnp.zeros((264,), jnp.float32)

    acc, cnt, self_rows = _sc_agg(
        inv_packed, src, dstt, nodes_perm, features, ones_in, zrow, zcnt)

    out_perm = _tc_combine(
        acc.reshape(B, R, D), cnt.reshape(B, R),
        self_rows, weight, relation_weights)

    canon = inv[nodes]
    return jnp.take(out_perm, (canon % 2) * HB + canon // 2, axis=0)


# speedup vs baseline: 1.2833x; 1.0733x over previous
import functools

import jax
import jax.numpy as jnp
from jax import lax
from jax.experimental import pallas as pl
from jax.experimental.pallas import tpu as pltpu
from jax.experimental.pallas import tpu_sc as plsc

N = 10000
E = 320000
R = 8
D = 128
B = 1024

NC = 2
NS = 16
NW = NC * NS
EPT = E // NS
SB = 2000
L = 16

CG = 64
HB = B // 2
NSEG = HB * R
DUMP = NSEG
NSEG_PAD = NSEG + 16
CPAD = 4224
CBUF = SB + 3 * CG
SROWS = B // NW


def _sc_body(inv_hbm, src_hbm, dstt_hbm, nodes_hbm, feat_hbm,
             ones_hbm, zrow_hbm, zcnt_hbm,
             acc_out, cnt_out, self_out,
             inv_v, src_v, dstt_v, src_w, dstt_w, csrc_v, cseg_v,
             src_row, seg_row, rows_v, onesb, zcnt_v, nidx_v,
             acc_sh, cnt_sh, sem, semb):
    c = lax.axis_index("c")
    s = lax.axis_index("s")
    wid = c * NS + s

    pltpu.sync_copy(inv_hbm, inv_v)
    pltpu.sync_copy(ones_hbm, onesb)
    pltpu.sync_copy(zrow_hbm, rows_v)
    pltpu.sync_copy(zcnt_hbm, zcnt_v)

    for q in range(4):
        pltpu.sync_copy(rows_v, acc_sh.at[pl.ds(s * 257 + q * CG, CG)])
    pltpu.sync_copy(rows_v.at[pl.ds(0, 1)], acc_sh.at[pl.ds(s * 257 + 256, 1)])
    pltpu.sync_copy(zcnt_v, cnt_sh.at[pl.ds(s * 264, 264)])
    plsc.subcore_barrier()

    def p1(i, cnt, dstt_v, src_v):
        base = i * L
        dstt16 = dstt_v[pl.ds(base, L)]
        src16 = src_v[pl.ds(base, L)]
        dst16 = lax.shift_right_logical(dstt16, 3)
        typ16 = dstt16 & 7
        slot16 = plsc.load_gather(inv_v, [dst16])
        m = (slot16 >= 0) & ((slot16 & 1) == c)
        seg16 = jnp.where(m, lax.shift_right_logical(slot16, 1) * R + typ16,
                          DUMP)
        plsc.store_compressed(csrc_v.at[pl.ds(cnt, L)], src16, mask=m)
        plsc.store_compressed(cseg_v.at[pl.ds(cnt, L)], seg16, mask=m)
        return cnt + jnp.sum(m.astype(jnp.int32))

    def p2(k, _):
        for j in range(CG // L):
            src_row[pl.ds(j * L, L)] = csrc_v[pl.ds(k * CG + j * L, L)]
            seg_row[pl.ds(j * L, L)] = cseg_v[pl.ds(k * CG + j * L, L)]
        pltpu.async_copy(feat_hbm.at[src_row], rows_v, sem).wait()
        pltpu.sync_copy(rows_v, acc_sh.at[seg_row], add=True)
        pltpu.sync_copy(onesb, cnt_sh.at[seg_row], add=True)
        return 0

    NT = EPT // SB
    ebufs = ((src_v, dstt_v, sem), (src_w, dstt_w, semb))

    def stage(t, b):
        sbuf, dbuf, sm = ebufs[b]
        off = s * EPT + t * SB
        return (pltpu.async_copy(src_hbm.at[pl.ds(off, SB)], sbuf, sm),
                pltpu.async_copy(dstt_hbm.at[pl.ds(off, SB)], dbuf, sm))

    def make_p1(b):
        sbuf, dbuf, _ = ebufs[b]
        def p1b(i, cnt):
            return p1(i, cnt, dbuf, sbuf)
        return p1b

    cnt = jnp.int32(0)
    descs = stage(0, 0)
    for t in range(NT):
        nxt = stage(t + 1, (t + 1) % 2) if t + 1 < NT else ()
        for d in descs:
            d.wait()
        cnt = lax.fori_loop(0, SB // L, make_p1(t % 2), cnt)
        nfull = cnt // CG
        lax.fori_loop(0, nfull, p2, 0)
        for j in range(CG // L):
            sv = csrc_v[pl.ds(nfull * CG + j * L, L)]
            gv = cseg_v[pl.ds(nfull * CG + j * L, L)]
            csrc_v[pl.ds(j * L, L)] = sv
            cseg_v[pl.ds(j * L, L)] = gv
        cnt = cnt - nfull * CG
        descs = nxt

    full = jnp.ones((L,), jnp.bool_)
    for k in range(CG // L):
        plsc.store_compressed(
            cseg_v.at[pl.ds(cnt + k * L, L)],
            jnp.full((L,), DUMP, jnp.int32), mask=full)
        plsc.store_compressed(
            csrc_v.at[pl.ds(cnt + k * L, L)],
            jnp.zeros((L,), jnp.int32), mask=full)
    lax.fori_loop(0, 1, p2, 0)
    plsc.subcore_barrier()

    pltpu.sync_copy(acc_sh.at[pl.ds(s * 256, 256)],
                    acc_out.at[c, pl.ds(s * 256, 256)])
    pltpu.sync_copy(cnt_sh.at[pl.ds(s * 256, 256)],
                    cnt_out.at[c, pl.ds(s * 256, 256)])

    pltpu.sync_copy(nodes_hbm.at[pl.ds(wid * SROWS, SROWS)], nidx_v)
    pltpu.async_copy(feat_hbm.at[nidx_v], rows_v.at[pl.ds(0, SROWS)],
                     sem).wait()
    pltpu.sync_copy(rows_v.at[pl.ds(0, SROWS)],
                    self_out.at[pl.ds(wid * SROWS, SROWS)])


_sc_agg = functools.partial(
    pl.kernel,
    out_type=[
        jax.ShapeDtypeStruct((NC, NSEG, D), jnp.float32),
        jax.ShapeDtypeStruct((NC, NSEG), jnp.float32),
        jax.ShapeDtypeStruct((B, D), jnp.float32),
    ],
    mesh=plsc.VectorSubcoreMesh(
        core_axis_name="c", subcore_axis_name="s",
        num_cores=NC, num_subcores=NS),
    scratch_types=[
        pltpu.VMEM((N,), jnp.int32),
        pltpu.VMEM((SB,), jnp.int32),
        pltpu.VMEM((SB,), jnp.int32),
        pltpu.VMEM((SB,), jnp.int32),
        pltpu.VMEM((SB,), jnp.int32),
        pltpu.VMEM((CBUF,), jnp.int32),
        pltpu.VMEM((CBUF,), jnp.int32),
        pltpu.VMEM((CG,), jnp.int32),
        pltpu.VMEM((CG,), jnp.int32),
        pltpu.VMEM((CG, D), jnp.float32),
        pltpu.VMEM((CG,), jnp.float32),
        pltpu.VMEM((264,), jnp.float32),
        pltpu.VMEM((SROWS,), jnp.int32),
        pltpu.VMEM_SHARED((NSEG_PAD, D), jnp.float32),
        pltpu.VMEM_SHARED((CPAD,), jnp.float32),
        pltpu.SemaphoreType.DMA,
        pltpu.SemaphoreType.DMA,
    ],
    compiler_params=pltpu.CompilerParams(needs_layout_passes=False),
)(_sc_body)


BT = 256


def _tc_body(acc_ref, cnt_ref, self_ref, w_ref, rw_ref, out_ref):
    acc = acc_ref[...]
    cnt = cnt_ref[...]
    rel = jnp.zeros((BT, D), jnp.float32)
    for r in range(R):
        mean_r = acc[:, r, :] / (cnt[:, r:r + 1] + 1e-10)
        rel = rel + lax.dot_general(
            mean_r, rw_ref[r],
            (((1,), (1,)), ((), ())), preferred_element_type=jnp.float32)
    self_o = lax.dot_general(
        self_ref[...], w_ref[...],
        (((1,), (1,)), ((), ())), preferred_element_type=jnp.float32)
    out_ref[...] = jnp.maximum(self_o + rel, 0.0)


def _tc_combine(acc3, cnt2, self_rows, weight, relation_weights):
    return pl.pallas_call(
        _tc_body,
        grid=(B // BT,),
        in_specs=[
            pl.BlockSpec((BT, R, D), lambda i: (i, 0, 0)),
            pl.BlockSpec((BT, R), lambda i: (i, 0)),
            pl.BlockSpec((BT, D), lambda i: (i, 0)),
            pl.BlockSpec((D, D), lambda i: (0, 0)),
            pl.BlockSpec((R, D, D), lambda i: (0, 0, 0)),
        ],
        out_specs=pl.BlockSpec((BT, D), lambda i: (i, 0)),
        out_shape=jax.ShapeDtypeStruct((B, D), jnp.float32),
    )(acc3, cnt2, self_rows, weight, relation_weights)


def kernel(nodes, features, edge_index, edge_type, weight, relation_weights):
    nodes = nodes.astype(jnp.int32)
    src = edge_index[0].astype(jnp.int32)
    dstt = edge_index[1].astype(jnp.int32) * 8 + edge_type.astype(jnp.int32)

    inv = jnp.full((N,), -1, jnp.int32).at[nodes].set(
        jnp.arange(B, dtype=jnp.int32))

    nodes_perm = jnp.transpose(nodes.reshape(HB, 2)).reshape(B)

    ones_in = jnp.ones((CG,), jnp.float32)
    zrow = jnp.zeros((CG, D), jnp.float32)
    zcnt = jnp.zeros((264,), jnp.float32)

    acc, cnt, self_rows = _sc_agg(
        inv, src, dstt, nodes_perm, features, ones_in, zrow, zcnt)

    out_perm = _tc_combine(
        acc.reshape(B, R, D), cnt.reshape(B, R),
        self_rows, weight, relation_weights)

    canon = inv[nodes]
    return jnp.take(out_perm, (canon % 2) * HB + canon // 2, axis=0)


# speedup vs baseline: 1.2995x; 1.0126x over previous
import functools

import jax
import jax.numpy as jnp
from jax import lax
from jax.experimental import pallas as pl
from jax.experimental.pallas import tpu as pltpu
from jax.experimental.pallas import tpu_sc as plsc

N = 10000
E = 320000
R = 8
D = 128
B = 1024

NC = 2
NS = 16
NW = NC * NS
EPT = E // NS
SB = 2000
L = 16

CG = 64
HB = B // 2
NSEG = HB * R
DUMP = NSEG
NSEG_PAD = NSEG + 16
CPAD = 4224
CBUF = SB + 3 * CG
SROWS = B // NW


def _sc_body(inv_hbm, src_hbm, dstt_hbm, nodes_hbm, feat_hbm,
             ones_hbm, zrow_hbm, zcnt_hbm,
             acc_out, cnt_out, self_out,
             inv_v, src_v, dstt_v, src_w, dstt_w, csrc_v, cseg_v,
             src_row, seg_row, rows_v, onesb, zcnt_v, nidx_v,
             acc_sh, cnt_sh, sem, semb, semc):
    c = lax.axis_index("c")
    s = lax.axis_index("s")
    wid = c * NS + s

    NT = EPT // SB

    def stage(t, b, sbuf, dbuf, sm):
        off = s * EPT + t * SB
        return (pltpu.async_copy(src_hbm.at[pl.ds(off, SB)], sbuf, sm),
                pltpu.async_copy(dstt_hbm.at[pl.ds(off, SB)], dbuf, sm))

    d_inv = pltpu.async_copy(inv_hbm, inv_v, semc)
    descs = stage(0, 0, src_v, dstt_v, sem)
    pltpu.sync_copy(ones_hbm, onesb)
    pltpu.sync_copy(zrow_hbm, rows_v)
    pltpu.sync_copy(zcnt_hbm, zcnt_v)

    for q in range(4):
        pltpu.sync_copy(rows_v, acc_sh.at[pl.ds(s * 257 + q * CG, CG)])
    pltpu.sync_copy(rows_v.at[pl.ds(0, 1)], acc_sh.at[pl.ds(s * 257 + 256, 1)])
    pltpu.sync_copy(zcnt_v, cnt_sh.at[pl.ds(s * 264, 264)])
    plsc.subcore_barrier()

    def p1(i, cnt, dstt_v, src_v):
        base = i * L
        dstt16 = dstt_v[pl.ds(base, L)]
        src16 = src_v[pl.ds(base, L)]
        dst16 = lax.shift_right_logical(dstt16, 3)
        typ16 = dstt16 & 7
        slot16 = plsc.load_gather(inv_v, [dst16])
        m = (slot16 >= 0) & ((slot16 & 1) == c)
        seg16 = jnp.where(m, lax.shift_right_logical(slot16, 1) * R + typ16,
                          DUMP)
        plsc.store_compressed(csrc_v.at[pl.ds(cnt, L)], src16, mask=m)
        plsc.store_compressed(cseg_v.at[pl.ds(cnt, L)], seg16, mask=m)
        return cnt + jnp.sum(m.astype(jnp.int32))

    def p2(k, _):
        for j in range(CG // L):
            src_row[pl.ds(j * L, L)] = csrc_v[pl.ds(k * CG + j * L, L)]
            seg_row[pl.ds(j * L, L)] = cseg_v[pl.ds(k * CG + j * L, L)]
        pltpu.async_copy(feat_hbm.at[src_row], rows_v, sem).wait()
        pltpu.sync_copy(rows_v, acc_sh.at[seg_row], add=True)
        pltpu.sync_copy(onesb, cnt_sh.at[seg_row], add=True)
        return 0

    ebufs = ((src_v, dstt_v, sem), (src_w, dstt_w, semb))

    def make_p1(b):
        sbuf, dbuf, _ = ebufs[b]
        def p1b(i, cnt):
            return p1(i, cnt, dbuf, sbuf)
        return p1b

    cnt = jnp.int32(0)
    d_inv.wait()
    for t in range(NT):
        nxt = (stage(t + 1, (t + 1) % 2, *ebufs[(t + 1) % 2])
               if t + 1 < NT else ())
        for d in descs:
            d.wait()
        cnt = lax.fori_loop(0, SB // L, make_p1(t % 2), cnt)
        nfull = cnt // CG
        lax.fori_loop(0, nfull, p2, 0)
        for j in range(CG // L):
            sv = csrc_v[pl.ds(nfull * CG + j * L, L)]
            gv = cseg_v[pl.ds(nfull * CG + j * L, L)]
            csrc_v[pl.ds(j * L, L)] = sv
            cseg_v[pl.ds(j * L, L)] = gv
        cnt = cnt - nfull * CG
        descs = nxt

    full = jnp.ones((L,), jnp.bool_)
    for k in range(CG // L):
        plsc.store_compressed(
            cseg_v.at[pl.ds(cnt + k * L, L)],
            jnp.full((L,), DUMP, jnp.int32), mask=full)
        plsc.store_compressed(
            csrc_v.at[pl.ds(cnt + k * L, L)],
            jnp.zeros((L,), jnp.int32), mask=full)
    lax.fori_loop(0, 1, p2, 0)
    plsc.subcore_barrier()

    pltpu.sync_copy(acc_sh.at[pl.ds(s * 256, 256)],
                    acc_out.at[c, pl.ds(s * 256, 256)])
    pltpu.sync_copy(cnt_sh.at[pl.ds(s * 256, 256)],
                    cnt_out.at[c, pl.ds(s * 256, 256)])

    pltpu.sync_copy(nodes_hbm.at[pl.ds(wid * SROWS, SROWS)], nidx_v)
    pltpu.async_copy(feat_hbm.at[nidx_v], rows_v.at[pl.ds(0, SROWS)],
                     sem).wait()
    pltpu.sync_copy(rows_v.at[pl.ds(0, SROWS)],
                    self_out.at[pl.ds(wid * SROWS, SROWS)])


_sc_agg = functools.partial(
    pl.kernel,
    out_type=[
        jax.ShapeDtypeStruct((NC, NSEG, D), jnp.float32),
        jax.ShapeDtypeStruct((NC, NSEG), jnp.float32),
        jax.ShapeDtypeStruct((B, D), jnp.float32),
    ],
    mesh=plsc.VectorSubcoreMesh(
        core_axis_name="c", subcore_axis_name="s",
        num_cores=NC, num_subcores=NS),
    scratch_types=[
        pltpu.VMEM((N,), jnp.int32),
        pltpu.VMEM((SB,), jnp.int32),
        pltpu.VMEM((SB,), jnp.int32),
        pltpu.VMEM((SB,), jnp.int32),
        pltpu.VMEM((SB,), jnp.int32),
        pltpu.VMEM((CBUF,), jnp.int32),
        pltpu.VMEM((CBUF,), jnp.int32),
        pltpu.VMEM((CG,), jnp.int32),
        pltpu.VMEM((CG,), jnp.int32),
        pltpu.VMEM((CG, D), jnp.float32),
        pltpu.VMEM((CG,), jnp.float32),
        pltpu.VMEM((264,), jnp.float32),
        pltpu.VMEM((SROWS,), jnp.int32),
        pltpu.VMEM_SHARED((NSEG_PAD, D), jnp.float32),
        pltpu.VMEM_SHARED((CPAD,), jnp.float32),
        pltpu.SemaphoreType.DMA,
        pltpu.SemaphoreType.DMA,
        pltpu.SemaphoreType.DMA,
    ],
    compiler_params=pltpu.CompilerParams(needs_layout_passes=False),
)(_sc_body)


BT = 256


def _tc_body(acc_ref, cnt_ref, self_ref, w_ref, rw_ref, out_ref):
    acc = acc_ref[...]
    cnt = cnt_ref[...]
    rel = jnp.zeros((BT, D), jnp.float32)
    for r in range(R):
        mean_r = acc[:, r, :] / (cnt[:, r:r + 1] + 1e-10)
        rel = rel + lax.dot_general(
            mean_r, rw_ref[r],
            (((1,), (1,)), ((), ())), preferred_element_type=jnp.float32)
    self_o = lax.dot_general(
        self_ref[...], w_ref[...],
        (((1,), (1,)), ((), ())), preferred_element_type=jnp.float32)
    out_ref[...] = jnp.maximum(self_o + rel, 0.0)


def _tc_combine(acc3, cnt2, self_rows, weight, relation_weights):
    return pl.pallas_call(
        _tc_body,
        grid=(B // BT,),
        in_specs=[
            pl.BlockSpec((BT, R, D), lambda i: (i, 0, 0)),
            pl.BlockSpec((BT, R), lambda i: (i, 0)),
            pl.BlockSpec((BT, D), lambda i: (i, 0)),
            pl.BlockSpec((D, D), lambda i: (0, 0)),
            pl.BlockSpec((R, D, D), lambda i: (0, 0, 0)),
        ],
        out_specs=pl.BlockSpec((BT, D), lambda i: (i, 0)),
        out_shape=jax.ShapeDtypeStruct((B, D), jnp.float32),
    )(acc3, cnt2, self_rows, weight, relation_weights)


def kernel(nodes, features, edge_index, edge_type, weight, relation_weights):
    nodes = nodes.astype(jnp.int32)
    src = edge_index[0].astype(jnp.int32)
    dstt = edge_index[1].astype(jnp.int32) * 8 + edge_type.astype(jnp.int32)

    inv = jnp.full((N,), -1, jnp.int32).at[nodes].set(
        jnp.arange(B, dtype=jnp.int32))

    nodes_perm = jnp.transpose(nodes.reshape(HB, 2)).reshape(B)

    ones_in = jnp.ones((CG,), jnp.float32)
    zrow = jnp.zeros((CG, D), jnp.float32)
    zcnt = jnp.zeros((264,), jnp.float32)

    acc, cnt, self_rows = _sc_agg(
        inv, src, dstt, nodes_perm, features, ones_in, zrow, zcnt)

    out_perm = _tc_combine(
        acc.reshape(B, R, D), cnt.reshape(B, R),
        self_rows, weight, relation_weights)

    canon = inv[nodes]
    return jnp.take(out_perm, (canon % 2) * HB + canon // 2, axis=0)
